# Initial kernel scaffold; baseline (speedup 1.0000x reference)
import jax
import jax.numpy as jnp
from jax.experimental import pallas as pl


def kernel(x, edge_index, W1, b1, g1, be1, W2, b2, g2, be2, W3, b3):
    raise NotImplementedError("write your pallas kernel here")


# speedup vs baseline: 14.3827x; 14.3827x over previous
import functools

import jax
import jax.numpy as jnp
from jax import lax
from jax.experimental import pallas as pl
from jax.experimental.pallas import tpu as pltpu
from jax.experimental.pallas import tpu_sc as plsc

N = 10000
E = 320000
H = 128
C = 40
CP = 128
EPS = 1e-5

NC = 2
NS = 16
NW = NC * NS
EPW = E // NW
B = 80
CHUNKS = EPW // B
NP = 10240
RPT = NP // NS

_MESH = plsc.VectorSubcoreMesh(core_axis_name="c", subcore_axis_name="s")


def _zero_rows(ref, nrows, d):
    dv = d // 16

    def body(i, _):
        r = i // dv
        col = (i % dv) * 16
        ref[r, pl.ds(col, 16)] = jnp.zeros((16,), jnp.float32)
        return ()

    lax.fori_loop(0, nrows * dv, body, ())


def _zero_acc_slice(acc_sh, zbuf, s):
    base = s * RPT

    def body(k, _):
        pltpu.sync_copy(zbuf, acc_sh.at[pl.ds(base + k * B, B)])
        return ()

    lax.fori_loop(0, RPT // B, body, ())


def _make_sc_edge_pass(d):
    @functools.partial(
        pl.kernel,
        out_type=jax.ShapeDtypeStruct((NC, NP, d), jnp.float32),
        mesh=_MESH,
        scratch_types=[
            pltpu.VMEM((CHUNKS, B), jnp.int32),
            pltpu.VMEM((CHUNKS, B), jnp.int32),
            pltpu.VMEM((B, d), jnp.float32),
            pltpu.VMEM_SHARED((NP, d), jnp.float32),
            pltpu.SemaphoreType.DMA,
        ],
    )
    def sc_edge_pass(y_hbm, src_hbm, dst_hbm, out_hbm, src_v, dst_v, rows_v,
                     acc_sh, sem):
        c = lax.axis_index("c")
        s = lax.axis_index("s")
        wid = c * NS + s
        pltpu.sync_copy(src_hbm.at[wid], src_v)
        pltpu.sync_copy(dst_hbm.at[wid], dst_v)
        _zero_rows(rows_v, B, d)
        _zero_acc_slice(acc_sh, rows_v, s)
        plsc.subcore_barrier()

        def chunk(j, _):
            pltpu.async_copy(y_hbm.at[src_v.at[j]], rows_v, sem).wait()
            pltpu.sync_copy(rows_v, acc_sh.at[dst_v.at[j]], add=True)
            return ()

        lax.fori_loop(0, CHUNKS, chunk, ())
        plsc.subcore_barrier()
        pltpu.sync_copy(acc_sh.at[pl.ds(s * RPT, RPT)],
                        out_hbm.at[c, pl.ds(s * RPT, RPT)])

    return sc_edge_pass


_sc_edge_pass_h = _make_sc_edge_pass(H)


@functools.partial(
    pl.kernel,
    out_type=jax.ShapeDtypeStruct((NC, NP, H), jnp.float32),
    mesh=_MESH,
    scratch_types=[
        pltpu.VMEM((CHUNKS, B), jnp.int32),
        pltpu.VMEM((B, H), jnp.float32),
        pltpu.VMEM_SHARED((NP, H), jnp.float32),
    ],
)
def _sc_degree(dst_hbm, out_hbm, dst_v, ones_v, acc_sh):
    c = lax.axis_index("c")
    s = lax.axis_index("s")
    wid = c * NS + s
    pltpu.sync_copy(dst_hbm.at[wid], dst_v)
    _zero_rows(ones_v, B, H)
    _zero_acc_slice(acc_sh, ones_v, s)

    dv = H // 16

    def fill(i, _):
        ones_v[i // dv, pl.ds((i % dv) * 16, 16)] = jnp.ones((16,), jnp.float32)
        return ()

    lax.fori_loop(0, B * dv, fill, ())
    plsc.subcore_barrier()

    def chunk(j, _):
        pltpu.sync_copy(ones_v, acc_sh.at[dst_v.at[j]], add=True)
        return ()

    lax.fori_loop(0, CHUNKS, chunk, ())
    plsc.subcore_barrier()
    pltpu.sync_copy(acc_sh.at[pl.ds(s * RPT, RPT)],
                    out_hbm.at[c, pl.ds(s * RPT, RPT)])


BN_TC = 1000


def _dinv_from(degp_ref):
    deg = degp_ref[0, :, 0] + degp_ref[1, :, 0] + 1.0
    return lax.rsqrt(deg)[:, None]


def _tc_in(x, w1, degp):
    def body(x_ref, w_ref, degp_ref, y_ref):
        dinv = _dinv_from(degp_ref)
        y_ref[...] = dinv * jnp.dot(x_ref[...], w_ref[...],
                                    preferred_element_type=jnp.float32)

    return pl.pallas_call(
        body,
        grid=(N // BN_TC,),
        in_specs=[
            pl.BlockSpec((BN_TC, H), lambda i: (i, 0)),
            pl.BlockSpec((H, H), lambda i: (0, 0)),
            pl.BlockSpec((NC, BN_TC, H), lambda i: (0, i, 0)),
        ],
        out_specs=pl.BlockSpec((BN_TC, H), lambda i: (i, 0)),
        out_shape=jax.ShapeDtypeStruct((N, H), jnp.float32),
    )(x, w1, degp)


def _tc_mid(accp, y, degp, w_next, g, be, b, res, dout):
    use_res = res is not None
    bn_s = 1.0 / (1.0 + EPS) ** 0.5

    def body(*refs):
        if use_res:
            a_ref, y_ref, degp_ref, w_ref, g_ref, be_ref, b_ref, r_ref, o_ref = refs
        else:
            a_ref, y_ref, degp_ref, w_ref, g_ref, be_ref, b_ref, o_ref = refs
        dinv = _dinv_from(degp_ref)
        conv = dinv * (a_ref[0] + a_ref[1] + y_ref[...]) + b_ref[...]
        h = conv * (g_ref[...] * bn_s) + be_ref[...]
        if use_res:
            h = h + r_ref[...]
        h = jnp.maximum(h, 0.0)
        o_ref[...] = dinv * jnp.dot(h, w_ref[...],
                                    preferred_element_type=jnp.float32)

    in_specs = [
        pl.BlockSpec((NC, BN_TC, H), lambda i: (0, i, 0)),
        pl.BlockSpec((BN_TC, H), lambda i: (i, 0)),
        pl.BlockSpec((NC, BN_TC, H), lambda i: (0, i, 0)),
        pl.BlockSpec((H, dout), lambda i: (0, 0)),
        pl.BlockSpec((1, H), lambda i: (0, 0)),
        pl.BlockSpec((1, H), lambda i: (0, 0)),
        pl.BlockSpec((1, H), lambda i: (0, 0)),
    ]
    args = [accp, y, degp, w_next, g, be, b]
    if use_res:
        in_specs.append(pl.BlockSpec((BN_TC, H), lambda i: (i, 0)))
        args.append(res)
    return pl.pallas_call(
        body,
        grid=(N // BN_TC,),
        in_specs=in_specs,
        out_specs=pl.BlockSpec((BN_TC, dout), lambda i: (i, 0)),
        out_shape=jax.ShapeDtypeStruct((N, dout), jnp.float32),
    )(*args)


def _tc_out(accp, y, degp, b3p):
    def body(a_ref, y_ref, degp_ref, b_ref, o_ref):
        dinv = _dinv_from(degp_ref)
        o = dinv * (a_ref[0] + a_ref[1] + y_ref[...]) + b_ref[...]
        col = lax.broadcasted_iota(jnp.int32, (BN_TC, CP), 1)
        o = jnp.where(col < C, o, -jnp.inf)
        m = jnp.max(o, axis=1, keepdims=True)
        z = o - m
        o_ref[...] = z - jnp.log(jnp.sum(jnp.exp(z), axis=1, keepdims=True))

    return pl.pallas_call(
        body,
        grid=(N // BN_TC,),
        in_specs=[
            pl.BlockSpec((NC, BN_TC, CP), lambda i: (0, i, 0)),
            pl.BlockSpec((BN_TC, CP), lambda i: (i, 0)),
            pl.BlockSpec((NC, BN_TC, H), lambda i: (0, i, 0)),
            pl.BlockSpec((1, CP), lambda i: (0, 0)),
        ],
        out_specs=pl.BlockSpec((BN_TC, CP), lambda i: (i, 0)),
        out_shape=jax.ShapeDtypeStruct((N, CP), jnp.float32),
    )(accp, y, degp, b3p)


def kernel(x, edge_index, W1, b1, g1, be1, W2, b2, g2, be2, W3, b3):
    src3 = edge_index[0].reshape(NW, CHUNKS, B)
    dst3 = edge_index[1].reshape(NW, CHUNKS, B)
    g1r, be1r, b1r = g1.reshape(1, H), be1.reshape(1, H), b1.reshape(1, H)
    g2r, be2r, b2r = g2.reshape(1, H), be2.reshape(1, H), b2.reshape(1, H)
    w3p = jnp.pad(W3, ((0, 0), (0, CP - C)))
    b3p = jnp.pad(b3, (0, CP - C)).reshape(1, CP)

    degp = _sc_degree(dst3)
    y1 = _tc_in(x, W1, degp)
    a1 = _sc_edge_pass_h(y1, src3, dst3)
    y2 = _tc_mid(a1, y1, degp, W2, g1r, be1r, b1r, x, H)
    a2 = _sc_edge_pass_h(y2, src3, dst3)
    y3 = _tc_mid(a2, y2, degp, w3p, g2r, be2r, b2r, None, CP)
    a3 = _sc_edge_pass_h(y3, src3, dst3)
    o = _tc_out(a3, y3, degp, b3p)
    return o[:, :C]
